# Initial kernel scaffold; baseline (speedup 1.0000x reference)
import jax
import jax.numpy as jnp
from jax import lax
from jax.experimental import pallas as pl
from jax.experimental.pallas import tpu as pltpu
from jax.experimental.pallas import tpu_sc as plsc

NC = 2
NS = 16
NW = NC * NS
CHUNK = 128
PADR = 112


def _sc_segsum(n_nodes, n_pad, ch, feat, with_hist):
  rows_per_tile = n_nodes // NS
  zrows = n_pad // NS

  mesh = plsc.VectorSubcoreMesh(
      core_axis_name="c", subcore_axis_name="s",
      num_cores=NC, num_subcores=NS)

  out_type = [jax.ShapeDtypeStruct((NC, n_nodes, feat), jnp.float32)]
  scratch = [
      pltpu.VMEM((ch, CHUNK), jnp.int32),
      pltpu.VMEM((ch, CHUNK), jnp.int32),
      pltpu.VMEM((2, CHUNK, feat), jnp.float32),
      pltpu.VMEM_SHARED((n_pad, feat), jnp.float32),
      pltpu.SemaphoreType.DMA,
      pltpu.SemaphoreType.DMA,
  ]
  if with_hist:
    out_type.append(jax.ShapeDtypeStruct((NW, n_nodes), jnp.float32))
    scratch.append(pltpu.VMEM((n_pad,), jnp.float32))

  def body(h_hbm, src_hbm, dst_hbm, s_out, *rest):
    if with_hist:
      hist_out, src_v, dst_v, buf, acc, sem0, sem1, hist_v = rest
    else:
      src_v, dst_v, buf, acc, sem0, sem1 = rest
      hist_out = hist_v = None
    cid = lax.axis_index("c")
    sid = lax.axis_index("s")
    wid = sid * NC + cid

    zero16 = jnp.zeros((16,), jnp.float32)

    def zrow(r, carry):
      def zcol(k, c2):
        buf[0, r, pl.ds(k * 16, 16)] = zero16
        return c2
      return lax.fori_loop(0, feat // 16, zcol, carry)
    lax.fori_loop(0, CHUNK, zrow, 0)

    base = sid * zrows
    full = zrows // CHUNK
    tail = zrows - full * CHUNK

    def zdma(i, carry):
      pltpu.sync_copy(buf.at[0],
                      acc.at[pl.ds(base + i * CHUNK, CHUNK)])
      return carry
    lax.fori_loop(0, full, zdma, 0)
    if tail:
      pltpu.sync_copy(buf.at[0, pl.ds(0, tail)],
                      acc.at[pl.ds(base + full * CHUNK, tail)])

    if with_hist:
      def zhist(i, carry):
        hist_v[pl.ds(i * 16, 16)] = zero16
        return carry
      lax.fori_loop(0, n_pad // 16, zhist, 0)

    plsc.subcore_barrier()

    pltpu.sync_copy(src_hbm.at[wid], src_v)
    pltpu.sync_copy(dst_hbm.at[wid], dst_v)

    ones16 = jnp.ones((16,), jnp.float32)

    def hist_chunk(j):
      def hstep(k, carry):
        idx = dst_v[j, pl.ds(k * 16, 16)]
        plsc.addupdate_scatter(hist_v, [idx], ones16)
        return carry
      lax.fori_loop(0, CHUNK // 16, hstep, 0)

    pltpu.async_copy(h_hbm.at[src_v.at[0]], buf.at[0], sem0)

    def pair(p, carry):
      j0 = 2 * p
      j1 = j0 + 1
      pltpu.make_async_copy(h_hbm.at[src_v.at[j0]], buf.at[0], sem0).wait()
      pltpu.async_copy(h_hbm.at[src_v.at[j1]], buf.at[1], sem1)
      if with_hist:
        hist_chunk(j0)
      pltpu.sync_copy(buf.at[0], acc.at[dst_v.at[j0]], add=True)
      pltpu.make_async_copy(h_hbm.at[src_v.at[j1]], buf.at[1], sem1).wait()

      @pl.when(j0 + 2 < ch)
      def _():
        pltpu.async_copy(h_hbm.at[src_v.at[j0 + 2]], buf.at[0], sem0)
      if with_hist:
        hist_chunk(j1)
      pltpu.sync_copy(buf.at[1], acc.at[dst_v.at[j1]], add=True)
      return carry
    lax.fori_loop(0, ch // 2, pair, 0)

    plsc.subcore_barrier()
    row0 = sid * rows_per_tile
    pltpu.sync_copy(acc.at[pl.ds(row0, rows_per_tile)],
                    s_out.at[cid, pl.ds(row0, rows_per_tile)])
    if with_hist:
      pltpu.sync_copy(hist_v.at[pl.ds(0, n_nodes)], hist_out.at[wid])

  return pl.kernel(body, out_type=tuple(out_type), mesh=mesh,
                   scratch_types=tuple(scratch))


def _tc_layer(n_nodes, feat, hidden, relu, blk=1000):
  def body(s_ref, hist_ref, h_ref, wl_ref, wr_ref, b_ref, o_ref):
    s = s_ref[0] + s_ref[1]
    cnt = jnp.sum(hist_ref[...], axis=0)
    mean = s * (1.0 / jnp.maximum(cnt, 1.0))[:, None]
    acc = jnp.dot(mean, wl_ref[...], preferred_element_type=jnp.float32)
    acc = acc + jnp.dot(h_ref[...], wr_ref[...],
                        preferred_element_type=jnp.float32)
    acc = acc + b_ref[...]
    if relu:
      acc = jnp.maximum(acc, 0.0)
    o_ref[...] = acc

  grid = (n_nodes // blk,)
  return pl.pallas_call(
      body,
      grid=grid,
      in_specs=[
          pl.BlockSpec((2, blk, feat), lambda j: (0, j, 0)),
          pl.BlockSpec((NW, blk), lambda j: (0, j)),
          pl.BlockSpec((blk, feat), lambda j: (j, 0)),
          pl.BlockSpec((feat, hidden), lambda j: (0, 0)),
          pl.BlockSpec((feat, hidden), lambda j: (0, 0)),
          pl.BlockSpec((1, hidden), lambda j: (0, 0)),
      ],
      out_specs=pl.BlockSpec((blk, hidden), lambda j: (j, 0)),
      out_shape=jax.ShapeDtypeStruct((n_nodes, hidden), jnp.float32),
  )


def kernel(x, edge_index, W1l, W1r, b1, W2l, W2r, b2):
  n, d = x.shape
  h = W1l.shape[0]
  e = edge_index.shape[1]
  ch = -(-e // (NW * CHUNK))
  ch += ch % 2
  e_pad = NW * ch * CHUNK
  pad = e_pad - e
  n_pad = n + PADR
  assert n % NS == 0 and n_pad % NS == 0 and d % 16 == 0

  src = edge_index[0]
  dst = edge_index[1]
  if pad:
    ar = jnp.arange(pad, dtype=jnp.int32)
    src = jnp.concatenate([src, ar % n])
    dst = jnp.concatenate([dst, n + ar % PADR])
  src_r = src.reshape(NW, ch, CHUNK)
  dst_r = dst.reshape(NW, ch, CHUNK)

  sc1 = _sc_segsum(n, n_pad, ch, d, with_hist=True)
  sc2 = _sc_segsum(n, n_pad, ch, h, with_hist=False)
  tc1 = _tc_layer(n, d, h, relu=True)
  tc2 = _tc_layer(n, h, h, relu=False)

  s1, hist = sc1(x, src_r, dst_r)
  h1 = tc1(s1, hist, x, W1l.T, W1r.T, b1[None, :])
  (s2,) = sc2(h1, src_r, dst_r)
  out = tc2(s2, hist, h1, W2l.T, W2r.T, b2[None, :])
  return out


# speedup vs baseline: 11.5169x; 11.5169x over previous
import jax
import jax.numpy as jnp
from jax import lax
from jax.experimental import pallas as pl
from jax.experimental.pallas import tpu as pltpu
from jax.experimental.pallas import tpu_sc as plsc

NC = 2
NS = 16
NW = NC * NS
CHUNK = 128
PADR = 240


def _sc_segsum(n_pad, ch, feat, with_cnt):
  zrows = n_pad // NS
  ib = 16
  assert ch % ib == 0 and ib % 2 == 0

  mesh = plsc.VectorSubcoreMesh(
      core_axis_name="c", subcore_axis_name="s",
      num_cores=NC, num_subcores=NS)

  out_type = [jax.ShapeDtypeStruct((NC, n_pad, feat), jnp.float32)]
  scratch = [
      pltpu.VMEM((ib, CHUNK), jnp.int32),
      pltpu.VMEM((ib, CHUNK), jnp.int32),
      pltpu.VMEM((2, CHUNK, feat), jnp.float32),
      pltpu.VMEM_SHARED((n_pad, feat), jnp.float32),
      pltpu.SemaphoreType.DMA,
      pltpu.SemaphoreType.DMA,
  ]
  if with_cnt:
    out_type.append(jax.ShapeDtypeStruct((NC * n_pad,), jnp.float32))
    scratch += [
        pltpu.VMEM((CHUNK,), jnp.float32),
        pltpu.VMEM((2048,), jnp.float32),
        pltpu.VMEM_SHARED((n_pad,), jnp.float32),
    ]

  def body(h_hbm, src_hbm, dst_hbm, s_out, *rest):
    if with_cnt:
      (cnt_out, src_v, dst_v, buf, acc, sem0, sem1,
       ones_v, zeros_v, cnt_acc) = rest
    else:
      src_v, dst_v, buf, acc, sem0, sem1 = rest
      cnt_out = ones_v = zeros_v = cnt_acc = None
    cid = lax.axis_index("c")
    sid = lax.axis_index("s")
    wid = sid * NC + cid

    zero16 = jnp.zeros((16,), jnp.float32)

    def zrow(r, carry):
      def zcol(k, c2):
        buf[0, r, pl.ds(k * 16, 16)] = zero16
        return c2
      return lax.fori_loop(0, feat // 16, zcol, carry)
    lax.fori_loop(0, CHUNK, zrow, 0)

    base = sid * zrows
    full = zrows // CHUNK
    tail = zrows - full * CHUNK

    def zdma(i, carry):
      pltpu.sync_copy(buf.at[0],
                      acc.at[pl.ds(base + i * CHUNK, CHUNK)])
      return carry
    lax.fori_loop(0, full, zdma, 0)
    if tail:
      pltpu.sync_copy(buf.at[0, pl.ds(0, tail)],
                      acc.at[pl.ds(base + full * CHUNK, tail)])

    if with_cnt:
      one16 = jnp.ones((16,), jnp.float32)
      for k in range(CHUNK // 16):
        ones_v[pl.ds(k * 16, 16)] = one16

      def zh(i, carry):
        zeros_v[pl.ds(i * 16, 16)] = zero16
        return carry
      lax.fori_loop(0, 2048 // 16, zh, 0)

      @pl.when(sid == 0)
      def _():
        nfull = n_pad // 2048
        def zc(i, carry):
          pltpu.sync_copy(zeros_v, cnt_acc.at[pl.ds(i * 2048, 2048)])
          return carry
        lax.fori_loop(0, nfull, zc, 0)
        ztail = n_pad - nfull * 2048
        if ztail:
          pltpu.sync_copy(zeros_v.at[pl.ds(0, ztail)],
                          cnt_acc.at[pl.ds(nfull * 2048, ztail)])

    plsc.subcore_barrier()

    def count_chunk(j):
      if with_cnt:
        pltpu.sync_copy(ones_v, cnt_acc.at[dst_v.at[j]], add=True)

    def group(g, carry):
      pltpu.sync_copy(src_hbm.at[wid, pl.ds(g * ib, ib)], src_v)
      pltpu.sync_copy(dst_hbm.at[wid, pl.ds(g * ib, ib)], dst_v)
      pltpu.async_copy(h_hbm.at[src_v.at[0]], buf.at[0], sem0)

      def pair(q, c2):
        j0 = 2 * q
        j1 = j0 + 1
        pltpu.make_async_copy(h_hbm.at[src_v.at[j0]], buf.at[0], sem0).wait()
        pltpu.async_copy(h_hbm.at[src_v.at[j1]], buf.at[1], sem1)
        pltpu.sync_copy(buf.at[0], acc.at[dst_v.at[j0]], add=True)
        count_chunk(j0)
        pltpu.make_async_copy(h_hbm.at[src_v.at[j1]], buf.at[1], sem1).wait()

        @pl.when(j0 + 2 < ib)
        def _():
          pltpu.async_copy(h_hbm.at[src_v.at[j0 + 2]], buf.at[0], sem0)
        pltpu.sync_copy(buf.at[1], acc.at[dst_v.at[j1]], add=True)
        count_chunk(j1)
        return c2
      lax.fori_loop(0, ib // 2, pair, 0)
      return carry
    lax.fori_loop(0, ch // ib, group, 0)

    plsc.subcore_barrier()
    pltpu.sync_copy(acc.at[pl.ds(base, zrows)],
                    s_out.at[cid, pl.ds(base, zrows)])
    if with_cnt:
      @pl.when(sid == 0)
      def _():
        pltpu.sync_copy(cnt_acc, cnt_out.at[pl.ds(cid * n_pad, n_pad)])

  return pl.kernel(body, out_type=tuple(out_type), mesh=mesh,
                   scratch_types=tuple(scratch))


def _tc_layer(n_pad, feat, hidden, relu, blk=1024):
  def body(s_ref, c_ref, h_ref, wl_ref, wr_ref, b_ref, o_ref):
    j = pl.program_id(0)
    s = s_ref[0] + s_ref[1]
    cnt = c_ref[0, pl.ds(j * blk, blk)] + c_ref[1, pl.ds(j * blk, blk)]
    mean = s * (1.0 / jnp.maximum(cnt, 1.0))[:, None]
    acc = jnp.dot(mean, wl_ref[...], preferred_element_type=jnp.float32)
    acc = acc + jnp.dot(h_ref[...], wr_ref[...],
                        preferred_element_type=jnp.float32)
    acc = acc + b_ref[...]
    if relu:
      acc = jnp.maximum(acc, 0.0)
    o_ref[...] = acc

  return pl.pallas_call(
      body,
      grid=(n_pad // blk,),
      in_specs=[
          pl.BlockSpec((2, blk, feat), lambda j: (0, j, 0)),
          pl.BlockSpec((2, n_pad), lambda j: (0, 0)),
          pl.BlockSpec((blk, feat), lambda j: (j, 0)),
          pl.BlockSpec((feat, hidden), lambda j: (0, 0)),
          pl.BlockSpec((feat, hidden), lambda j: (0, 0)),
          pl.BlockSpec((1, hidden), lambda j: (0, 0)),
      ],
      out_specs=pl.BlockSpec((blk, hidden), lambda j: (j, 0)),
      out_shape=jax.ShapeDtypeStruct((n_pad, hidden), jnp.float32),
  )


def kernel(x, edge_index, W1l, W1r, b1, W2l, W2r, b2):
  n, d = x.shape
  h = W1l.shape[0]
  e = edge_index.shape[1]
  ch = -(-e // (NW * CHUNK))
  ch += ch % 2
  e_pad = NW * ch * CHUNK
  pad = e_pad - e
  n_pad = n + PADR
  assert n % NS == 0 and n_pad % NS == 0 and d % 16 == 0

  src = edge_index[0]
  dst = edge_index[1]
  if pad:
    ar = jnp.arange(pad, dtype=jnp.int32)
    src = jnp.concatenate([src, ar % n])
    dst = jnp.concatenate([dst, n + ar % PADR])
  src_r = src.reshape(NW, ch, CHUNK)
  dst_r = dst.reshape(NW, ch, CHUNK)
  x_p = jnp.pad(x, ((0, PADR), (0, 0)))

  s1, cnt = _sc_segsum(n_pad, ch, d, with_cnt=True)(x, src_r, dst_r)
  cnt2 = cnt.reshape(NC, n_pad)
  h1 = _tc_layer(n_pad, d, h, relu=True)(
      s1, cnt2, x_p, W1l.T, W1r.T, b1[None, :])
  (s2,) = _sc_segsum(n_pad, ch, h, with_cnt=False)(h1, src_r, dst_r)
  out = _tc_layer(n_pad, h, h, relu=False)(
      s2, cnt2, h1, W2l.T, W2r.T, b2[None, :])
  return out[:n]
